# baseline (speedup 1.0000x reference)
import functools

import jax
import jax.numpy as jnp
from jax import lax
from jax.experimental import pallas as pl
from jax.experimental.pallas import tpu as pltpu
from jax.experimental.pallas import tpu_sc as plsc

_BATCH = 4096
_SEQ = 200
_D = 64
_NC = 2
_NS = 16
_NW = _NC * _NS

_ROWS = _BATCH * _SEQ
_CHUNK = _SEQ
_NCHUNKS = _ROWS // _CHUNK
_CPW = _NCHUNKS // _NW
_PAIRS = _CPW // 2
_TPAD = 72
_OPAD = 128
_SPLITS = ((0, 104), (104, 96))


def _body(idx_hbm, table_hbm, pos_hbm, out_hbm,
          idx_v, g_v, rows_v, pos_v, g0, g1, i0, i1, o0, o1):
    wid = lax.axis_index("s") * _NC + lax.axis_index("c")
    first = wid * _CPW

    pltpu.sync_copy(pos_hbm, pos_v)

    def start_idx(chunk, slot, sem):
        pltpu.async_copy(idx_hbm.at[chunk], idx_v.at[slot], sem)

    def wait_idx(slot, sem):
        pltpu.make_async_copy(idx_hbm.at[0], idx_v.at[slot], sem).wait()

    def start_gathers(slot, sem):
        for off, num in _SPLITS:
            pltpu.async_copy(
                table_hbm.at[idx_v.at[slot, pl.ds(off, num)]],
                g_v.at[slot, pl.ds(off, num)],
                sem,
            )

    def wait_gathers(slot, sem):
        for off, num in _SPLITS:
            pltpu.make_async_copy(
                table_hbm.at[idx_v.at[slot, pl.ds(off, num)]],
                g_v.at[slot, pl.ds(off, num)],
                sem,
            ).wait()

    def start_out(chunk, slot, sem):
        pltpu.async_copy(rows_v.at[slot],
                         out_hbm.at[pl.ds(chunk * _CHUNK, _CHUNK)], sem)

    def wait_out(slot, sem):
        pltpu.make_async_copy(rows_v.at[slot],
                              out_hbm.at[pl.ds(0, _CHUNK)], sem).wait()

    def add_pos(slot):
        def add_row(r, acc):
            for col in range(_D // 16):
                sl = pl.ds(col * 16, 16)
                rows_v[slot, r, sl] = g_v[slot, r, sl] + pos_v[r, sl]
            return acc

        lax.fori_loop(0, _CHUNK, add_row, 0, unroll=4)

    pltpu.sync_copy(idx_hbm.at[first], idx_v.at[0])
    start_gathers(0, g0)
    start_idx(first + 1, 1, i1)

    def pair_body(i, acc):
        a = first + 2 * i
        b = a + 1
        not_first = i > 0
        not_last = i < _PAIRS - 1

        @pl.when(not_first)
        def _():
            wait_out(1, o1)
        wait_idx(1, i1)
        start_gathers(1, g1)
        wait_gathers(0, g0)

        @pl.when(not_last)
        def _():
            start_idx(a + 2, 0, i0)
        @pl.when(not_first)
        def _():
            wait_out(0, o0)
        add_pos(0)
        start_out(a, 0, o0)

        wait_gathers(1, g1)

        @pl.when(not_last)
        def _():
            start_idx(a + 3, 1, i1)
            wait_idx(0, i0)
            start_gathers(0, g0)
        add_pos(1)
        start_out(b, 1, o1)
        return acc

    lax.fori_loop(0, _PAIRS, pair_body, 0)

    wait_out(0, o0)
    wait_out(1, o1)


@jax.jit
def _run(idx, token_table, position_table):
    mesh = plsc.VectorSubcoreMesh(core_axis_name="c", subcore_axis_name="s")
    grid_kernel = functools.partial(
        pl.kernel,
        mesh=mesh,
        out_type=jax.ShapeDtypeStruct((_ROWS, _OPAD), jnp.float32),
        scratch_types=[
            pltpu.VMEM((2, _CHUNK), jnp.int32),
            pltpu.VMEM((2, _CHUNK, _TPAD), jnp.float32),
            pltpu.VMEM((2, _CHUNK, _OPAD), jnp.float32),
            pltpu.VMEM((_CHUNK, _D), jnp.float32),
            pltpu.SemaphoreType.DMA,
            pltpu.SemaphoreType.DMA,
            pltpu.SemaphoreType.DMA,
            pltpu.SemaphoreType.DMA,
            pltpu.SemaphoreType.DMA,
            pltpu.SemaphoreType.DMA,
        ],
        compiler_params=pltpu.CompilerParams(use_tc_tiling_on_sc=False),
    )(_body)
    out = grid_kernel(idx, token_table, position_table)
    return out[:, :_D].reshape(_BATCH, _SEQ, _D)


def kernel(inputs, token_table, position_table):
    padded = jnp.pad(token_table, ((0, 0), (0, _TPAD - _D)))
    idx = inputs.astype(jnp.int32).reshape(_NCHUNKS, _CHUNK)
    return _run(idx, padded, position_table)


# speedup vs baseline: 1.5438x; 1.5438x over previous
import functools

import jax
import jax.numpy as jnp
from jax import lax
from jax.experimental import pallas as pl
from jax.experimental.pallas import tpu as pltpu
from jax.experimental.pallas import tpu_sc as plsc

_BATCH = 4096
_SEQ = 200
_D = 64
_NC = 2
_NS = 16
_NW = _NC * _NS

_ROWS = _BATCH * _SEQ
_CHUNK_SEQ = 2
_CHUNK = _CHUNK_SEQ * _SEQ
_NCHUNKS = _ROWS // _CHUNK
_CPW = _NCHUNKS // _NW
_PAIRS = _CPW // 2
_PAD = 128
_SPLITS = ((0, 104), (104, 96))


def _body(idx_hbm, table_hbm, pos_hbm, out_hbm,
          idx_v, rows_v, pos_v, g0, g1, i0, i1, o0, o1):
    wid = lax.axis_index("s") * _NC + lax.axis_index("c")
    first = wid * _CPW

    for t in range(_CHUNK_SEQ):
        pltpu.sync_copy(pos_hbm, pos_v.at[pl.ds(t * _SEQ, _SEQ)])

    def start_idx(chunk, slot, sem):
        pltpu.async_copy(idx_hbm.at[pl.ds(chunk * _CHUNK_SEQ, _CHUNK_SEQ)],
                         idx_v.at[slot], sem)

    def wait_idx(slot, sem):
        pltpu.make_async_copy(idx_hbm.at[pl.ds(0, _CHUNK_SEQ)],
                              idx_v.at[slot], sem).wait()

    def start_gathers(slot, sem):
        for t in range(_CHUNK_SEQ):
            for off, num in _SPLITS:
                pltpu.async_copy(
                    table_hbm.at[idx_v.at[slot, t, pl.ds(off, num)]],
                    rows_v.at[slot, pl.ds(t * _SEQ + off, num)],
                    sem,
                )

    def wait_gathers(slot, sem):
        for t in range(_CHUNK_SEQ):
            for off, num in _SPLITS:
                pltpu.make_async_copy(
                    table_hbm.at[idx_v.at[slot, t, pl.ds(off, num)]],
                    rows_v.at[slot, pl.ds(t * _SEQ + off, num)],
                    sem,
                ).wait()

    def start_out(chunk, slot, sem):
        pltpu.async_copy(rows_v.at[slot],
                         out_hbm.at[pl.ds(chunk * _CHUNK, _CHUNK)], sem)

    def wait_out(slot, sem):
        pltpu.make_async_copy(rows_v.at[slot],
                              out_hbm.at[pl.ds(0, _CHUNK)], sem).wait()

    def add_pos(slot):
        def add_row(r, acc):
            for col in range(_D // 16):
                sl = pl.ds(col * 16, 16)
                plsc.addupdate(rows_v.at[slot, r, sl], pos_v[r, sl])
            return acc

        lax.fori_loop(0, _CHUNK, add_row, 0, unroll=4)

    pltpu.sync_copy(idx_hbm.at[pl.ds(first * _CHUNK_SEQ, _CHUNK_SEQ)], idx_v.at[0])
    start_gathers(0, g0)
    start_idx(first + 1, 1, i1)

    def pair_body(i, acc):
        a = first + 2 * i
        b = a + 1
        not_first = i > 0
        not_last = i < _PAIRS - 1

        @pl.when(not_first)
        def _():
            wait_out(1, o1)
        wait_idx(1, i1)
        start_gathers(1, g1)
        wait_gathers(0, g0)

        @pl.when(not_last)
        def _():
            start_idx(a + 2, 0, i0)
        add_pos(0)
        start_out(a, 0, o0)

        wait_gathers(1, g1)

        @pl.when(not_last)
        def _():
            start_idx(a + 3, 1, i1)
        add_pos(1)

        @pl.when(not_last)
        def _():
            wait_out(0, o0)
            wait_idx(0, i0)
            start_gathers(0, g0)
        start_out(b, 1, o1)
        return acc

    lax.fori_loop(0, _PAIRS, pair_body, 0)

    wait_out(0, o0)
    wait_out(1, o1)


@jax.jit
def _run(idx, token_table, position_table):
    mesh = plsc.VectorSubcoreMesh(core_axis_name="c", subcore_axis_name="s")
    grid_kernel = functools.partial(
        pl.kernel,
        mesh=mesh,
        out_type=jax.ShapeDtypeStruct((_ROWS, _PAD), jnp.float32),
        scratch_types=[
            pltpu.VMEM((2, _CHUNK_SEQ, _SEQ), jnp.int32),
            pltpu.VMEM((2, _CHUNK, _PAD), jnp.float32),
            pltpu.VMEM((_CHUNK, _D), jnp.float32),
            pltpu.SemaphoreType.DMA,
            pltpu.SemaphoreType.DMA,
            pltpu.SemaphoreType.DMA,
            pltpu.SemaphoreType.DMA,
            pltpu.SemaphoreType.DMA,
            pltpu.SemaphoreType.DMA,
        ],
        compiler_params=pltpu.CompilerParams(use_tc_tiling_on_sc=False),
    )(_body)
    out = grid_kernel(idx, token_table, position_table)
    return out[:, :_D].reshape(_BATCH, _SEQ, _D)


def kernel(inputs, token_table, position_table):
    padded = jnp.pad(token_table, ((0, 0), (0, _PAD - _D)))
    return _run(inputs.astype(jnp.int32), padded, position_table)
